# Initial kernel scaffold; baseline (speedup 1.0000x reference)
import jax
import jax.numpy as jnp
from jax.experimental import pallas as pl


def kernel(user_emb, item_emb, rows, cols, vals, noise, users, items):
    raise NotImplementedError("write your pallas kernel here")


# speedup vs baseline: 5.0694x; 5.0694x over previous
import jax
import jax.numpy as jnp
from jax import lax
from jax.experimental import pallas as pl
from jax.experimental.pallas import tpu as pltpu
from jax.experimental.pallas import tpu_sc as plsc

_N_USERS = 20000
_N_ITEMS = 30000
_N_NODES = _N_USERS + _N_ITEMS
_EMB = 64
_NNZ = 800000
_BATCH = 4096
_EPS = 0.2

_NC = 2
_NS = 16
_HALF = _EMB // _NC
_N_PAD = 50048
_RPT = _N_PAD // _NS
_RCH = 136
_NZCH = 23
_EPT = 50048
_NNZ_PAD = _NS * _EPT
_CH = 128
_GRP = 17
_NGRP = 23
_GEDGES = _GRP * _CH


def _noise_body(noise_ref, out_ref):
    x = noise_ref[...]
    ss = jnp.sum(x * x, axis=-1, keepdims=True)
    out_ref[...] = x * (_EPS / jnp.maximum(jnp.sqrt(ss), 1e-12))


def _noise_norm(noise):
    return pl.pallas_call(
        _noise_body,
        grid=(3, 125),
        in_specs=[pl.BlockSpec((1, 400, _EMB), lambda k, j: (k, j, 0))],
        out_specs=pl.BlockSpec((1, 400, _EMB), lambda k, j: (k, j, 0)),
        out_shape=jax.ShapeDtypeStruct(noise.shape, jnp.float32),
    )(noise)


def _sc_body(x0, rows, cols, vals, nn, unodes, inodes,
             e0, e1, e2, e3, e4, e5, ua, ia, ut, it, uc, ic,
             acc, mcols, mrows, mvals, idx_v, row_v, gat, prod,
             tbuf, nbuf, zbuf):
    c = lax.axis_index("c")
    s = lax.axis_index("s")
    coff = c * _N_PAD
    rbase = s * _RPT
    ebase = s * _EPT
    third = jnp.float32(1.0 / 3.0)
    ebufs = [e0, e1, e2, e3, e4, e5]
    srcs = [x0, e0, e1, e2, e3, e4]

    zv = jnp.zeros((16,), jnp.float32)

    def _zb(rr, _):
        for u in range(8):
            r = rr * 8 + u
            zbuf[r, pl.ds(0, 16)] = zv
            zbuf[r, pl.ds(16, 16)] = zv
        return 0
    lax.fori_loop(0, _RCH // 8, _zb, 0)

    def _z0(z, _):
        pltpu.sync_copy(zbuf, acc.at[pl.ds(rbase + z * _RCH, _RCH)])
        return 0
    lax.fori_loop(0, _NZCH, _z0, 0)
    plsc.subcore_barrier()

    def _vacc(dst_buf, src_buf):
        def body(t, _):
            for u in range(8):
                r = t * 8 + u
                for h in (0, 16):
                    dst_buf[r, pl.ds(h, 16)] = (
                        dst_buf[r, pl.ds(h, 16)] + src_buf[r, pl.ds(h, 16)])
            return 0
        lax.fori_loop(0, 16, body, 0)

    def _vscale(buf, sc):
        def body(t, _):
            for u in range(8):
                r = t * 8 + u
                for h in (0, 16):
                    buf[r, pl.ds(h, 16)] = buf[r, pl.ds(h, 16)] * sc
            return 0
        lax.fori_loop(0, 16, body, 0)

    for k in range(6):
        src = srcs[k]
        dst = ebufs[k]

        def _chunk(i, _):
            eb = i * _CH
            for j in range(8):
                idx_v[pl.ds(j * 16, 16)] = (
                    mcols[pl.ds(eb + j * 16, 16)] + coff)
                row_v[pl.ds(j * 16, 16)] = mrows[pl.ds(eb + j * 16, 16)]
            pltpu.sync_copy(src.at[idx_v], gat)

            def _edge16(t, _):
                b = t * 16
                val16 = mvals[pl.ds(eb + b, 16)]
                for u in range(16):
                    e = b + u
                    v = val16[u]
                    prod[e, pl.ds(0, 16)] = gat[e, pl.ds(0, 16)] * v
                    prod[e, pl.ds(16, 16)] = gat[e, pl.ds(16, 16)] * v
                return 0
            lax.fori_loop(0, 8, _edge16, 0)
            pltpu.sync_copy(prod, acc.at[row_v], add=True)
            return 0

        def _group(g, _):
            base = ebase + g * _GEDGES
            pltpu.sync_copy(cols.at[pl.ds(base, _GEDGES)], mcols)
            pltpu.sync_copy(rows.at[pl.ds(base, _GEDGES)], mrows)
            pltpu.sync_copy(vals.at[pl.ds(base, _GEDGES)], mvals)
            lax.fori_loop(0, _GRP, _chunk, 0)
            return 0
        lax.fori_loop(0, _NGRP, _group, 0)
        plsc.subcore_barrier()

        if k < 3:
            pltpu.sync_copy(acc.at[pl.ds(rbase, _RPT)],
                            dst.at[pl.ds(coff + rbase, _RPT)])

            def _zl(z, _):
                pltpu.sync_copy(zbuf, acc.at[pl.ds(rbase + z * _RCH, _RCH)])
                return 0
            lax.fori_loop(0, _NZCH, _zl, 0)
        else:
            def _ep(z, _):
                r0 = rbase + z * _RCH
                pltpu.sync_copy(acc.at[pl.ds(r0, _RCH)], tbuf)
                pltpu.sync_copy(nn.at[k - 3, pl.ds(coff + r0, _RCH)], nbuf)

                def _ew(rr, _):
                    for u in range(8):
                        r = rr * 8 + u
                        for h in (0, 16):
                            t = tbuf[r, pl.ds(h, 16)]
                            nb = nbuf[r, pl.ds(h, 16)]
                            tbuf[r, pl.ds(h, 16)] = t + jnp.sign(t) * nb
                    return 0
                lax.fori_loop(0, _RCH // 8, _ew, 0)
                pltpu.sync_copy(tbuf, dst.at[pl.ds(coff + r0, _RCH)])
                pltpu.sync_copy(zbuf, acc.at[pl.ds(r0, _RCH)])
                return 0
            lax.fori_loop(0, _NZCH, _ep, 0)
        plsc.subcore_barrier()

    def _emit(nodes_hbm, out_a, out_t, out_c):
        for z in range(2):
            b0 = s * 256 + z * _CH
            pltpu.sync_copy(nodes_hbm.at[pl.ds(b0, _CH)], row_v)
            for j in range(8):
                idx_v[pl.ds(j * 16, 16)] = row_v[pl.ds(j * 16, 16)] + coff
            pltpu.sync_copy(ebufs[0].at[idx_v], prod)
            for kk in (1, 2):
                pltpu.sync_copy(ebufs[kk].at[idx_v], gat)
                _vacc(prod, gat)
            _vscale(prod, third)
            pltpu.sync_copy(prod, out_a.at[c, pl.ds(b0, _CH)])
            pltpu.sync_copy(ebufs[3].at[idx_v], prod)
            pltpu.sync_copy(prod, out_c.at[c, pl.ds(b0, _CH)])
            for kk in (4, 5):
                pltpu.sync_copy(ebufs[kk].at[idx_v], gat)
                _vacc(prod, gat)
            _vscale(prod, third)
            pltpu.sync_copy(prod, out_t.at[c, pl.ds(b0, _CH)])

    _emit(unodes, ua, ut, uc)
    _emit(inodes, ia, it, ic)


def _sc_call():
    f32 = jnp.float32
    ebuf = jax.ShapeDtypeStruct((_NC * _N_PAD, _HALF), f32)
    outb = jax.ShapeDtypeStruct((_NC, _BATCH, _HALF), f32)
    return pl.kernel(
        _sc_body,
        out_type=[ebuf] * 6 + [outb] * 6,
        mesh=plsc.VectorSubcoreMesh(core_axis_name="c", subcore_axis_name="s",
                                    num_cores=_NC, num_subcores=_NS),
        compiler_params=pltpu.CompilerParams(use_tc_tiling_on_sc=False),
        scratch_types=[
            pltpu.VMEM_SHARED((_N_PAD, _HALF), f32),
            pltpu.VMEM((_GEDGES,), jnp.int32),
            pltpu.VMEM((_GEDGES,), jnp.int32),
            pltpu.VMEM((_GEDGES,), f32),
            pltpu.VMEM((_CH,), jnp.int32),
            pltpu.VMEM((_CH,), jnp.int32),
            pltpu.VMEM((_CH, _HALF), f32),
            pltpu.VMEM((_CH, _HALF), f32),
            pltpu.VMEM((_RCH, _HALF), f32),
            pltpu.VMEM((_RCH, _HALF), f32),
            pltpu.VMEM((_RCH, _HALF), f32),
        ],
    )


def kernel(user_emb, item_emb, rows, cols, vals, noise, users, items):
    nn = _noise_norm(noise)
    nn_flat = jnp.pad(nn, ((0, 0), (0, _N_PAD - _N_NODES), (0, 0)))
    nn_flat = nn_flat.reshape(3, _N_PAD, _NC, _HALF).transpose(0, 2, 1, 3)
    nn_flat = nn_flat.reshape(3, _NC * _N_PAD, _HALF)

    ego = jnp.concatenate([user_emb, item_emb], axis=0)
    x0 = jnp.pad(ego, ((0, _N_PAD - _N_NODES), (0, 0)))
    x0 = x0.reshape(_N_PAD, _NC, _HALF).transpose(1, 0, 2)
    x0 = x0.reshape(_NC * _N_PAD, _HALF)

    pad_e = _NNZ_PAD - _NNZ
    rows_p = jnp.pad(rows, (0, pad_e))
    cols_p = jnp.pad(cols, (0, pad_e))
    vals_p = jnp.pad(vals, (0, pad_e))

    inodes = items + _N_USERS

    res = _sc_call()(x0, rows_p, cols_p, vals_p, nn_flat, users, inodes)
    ua, ia, ut, it, uc, ic = res[6:]

    def fix(a):
        return a.transpose(1, 0, 2).reshape(_BATCH, _EMB)
    return (fix(ua), fix(ia), fix(ut), fix(it), fix(uc), fix(ic))
